# baseline (speedup 1.0000x reference)
import functools

import jax
import jax.numpy as jnp
from jax import lax
from jax.experimental import pallas as pl
from jax.experimental.pallas import tpu as pltpu
from jax.experimental.pallas import tpu_sc as plsc

N = 10000
NP = 10240
E = 320000
C = 128
G = 64
NEG_SLOPE = 0.2

BN = 2048
NC = 2
NS = 16
NW = NC * NS
KE = 32
NCHUNK = E // KE
CPW = -(-NCHUNK // NW)

_HIGH = jax.lax.Precision.HIGHEST


def _dense_first_body(x_ref, w_ref, att_ref, wd_ref, h_ref, as_ref, ad_ref):
    x = x_ref[...]
    h = jnp.dot(x, w_ref[...], preferred_element_type=jnp.float32,
                precision=_HIGH)
    h_ref[...] = h
    as_ref[...] = jnp.dot(h, att_ref[...], preferred_element_type=jnp.float32,
                          precision=_HIGH)
    ad_ref[...] = jnp.dot(x, wd_ref[...], preferred_element_type=jnp.float32,
                          precision=_HIGH)


def _epilogue(acc_ref, den_ref, b_ref, i):
    acc = acc_ref[0] + acc_ref[1]
    den = jnp.sum(den_ref[:, pl.ds(i * BN, BN)], axis=0)
    return jnp.maximum(acc / (den[:, None] + 1e-16) + b_ref[...], 0.0)


def _dense_mid_body(acc_ref, den_ref, b_ref, w_ref, att_ref, wd_ref,
                    h_ref, as_ref, ad_ref):
    x = _epilogue(acc_ref, den_ref, b_ref, pl.program_id(0))
    h = jnp.dot(x, w_ref[...], preferred_element_type=jnp.float32,
                precision=_HIGH)
    h_ref[...] = h
    as_ref[...] = jnp.dot(h, att_ref[...], preferred_element_type=jnp.float32,
                          precision=_HIGH)
    ad_ref[...] = jnp.dot(x, wd_ref[...], preferred_element_type=jnp.float32,
                          precision=_HIGH)


def _pool_body(acc_ref, den_ref, b_ref, batch_ref, out_ref, cnt_ref):
    i = pl.program_id(0)
    h = _epilogue(acc_ref, den_ref, b_ref, i)
    bids = batch_ref[...]
    iota = lax.broadcasted_iota(jnp.int32, (BN, G), 1).astype(jnp.float32)
    mask = (iota == bids).astype(jnp.float32)
    dnums = (((0,), (0,)), ((), ()))
    part = lax.dot_general(mask, h, dnums, preferred_element_type=jnp.float32,
                           precision=_HIGH)
    pcnt = lax.dot_general(mask, jnp.ones((BN, 1), jnp.float32), dnums,
                           preferred_element_type=jnp.float32,
                           precision=_HIGH)

    @pl.when(i == 0)
    def _():
        out_ref[...] = jnp.zeros_like(out_ref)
        cnt_ref[...] = jnp.zeros_like(cnt_ref)

    out_ref[...] += part
    cnt_ref[...] += pcnt

    @pl.when(i == pl.num_programs(0) - 1)
    def _():
        out_ref[...] = out_ref[...] / jnp.clip(cnt_ref[...], 1.0, None)


def _dense_first(x, W, att, wd):
    return pl.pallas_call(
        _dense_first_body,
        grid=(NP // BN,),
        in_specs=[
            pl.BlockSpec((BN, C), lambda i: (i, 0)),
            pl.BlockSpec((C, C), lambda i: (0, 0)),
            pl.BlockSpec((C, 1), lambda i: (0, 0)),
            pl.BlockSpec((C, 1), lambda i: (0, 0)),
        ],
        out_specs=[
            pl.BlockSpec((BN, C), lambda i: (i, 0)),
            pl.BlockSpec((BN, 1), lambda i: (i, 0)),
            pl.BlockSpec((BN, 1), lambda i: (i, 0)),
        ],
        out_shape=[
            jax.ShapeDtypeStruct((NP, C), jnp.float32),
            jax.ShapeDtypeStruct((NP, 1), jnp.float32),
            jax.ShapeDtypeStruct((NP, 1), jnp.float32),
        ],
    )(x, W, att[:, None], wd[:, None])


def _dense_mid(acc, den, b, W, att, wd):
    return pl.pallas_call(
        _dense_mid_body,
        grid=(NP // BN,),
        in_specs=[
            pl.BlockSpec((NC, BN, C), lambda i: (0, i, 0)),
            pl.BlockSpec((NW, NP), lambda i: (0, 0)),
            pl.BlockSpec((1, C), lambda i: (0, 0)),
            pl.BlockSpec((C, C), lambda i: (0, 0)),
            pl.BlockSpec((C, 1), lambda i: (0, 0)),
            pl.BlockSpec((C, 1), lambda i: (0, 0)),
        ],
        out_specs=[
            pl.BlockSpec((BN, C), lambda i: (i, 0)),
            pl.BlockSpec((BN, 1), lambda i: (i, 0)),
            pl.BlockSpec((BN, 1), lambda i: (i, 0)),
        ],
        out_shape=[
            jax.ShapeDtypeStruct((NP, C), jnp.float32),
            jax.ShapeDtypeStruct((NP, 1), jnp.float32),
            jax.ShapeDtypeStruct((NP, 1), jnp.float32),
        ],
    )(acc, den, b[None, :], W, att[:, None], wd[:, None])


def _pool(acc, den, b, batch_f):
    return pl.pallas_call(
        _pool_body,
        grid=(NP // BN,),
        in_specs=[
            pl.BlockSpec((NC, BN, C), lambda i: (0, i, 0)),
            pl.BlockSpec((NW, NP), lambda i: (0, 0)),
            pl.BlockSpec((1, C), lambda i: (0, 0)),
            pl.BlockSpec((BN, 1), lambda i: (i, 0)),
        ],
        out_specs=pl.BlockSpec((G, C), lambda i: (0, 0)),
        out_shape=jax.ShapeDtypeStruct((G, C), jnp.float32),
        scratch_shapes=[pltpu.VMEM((G, 1), jnp.float32)],
    )(acc, den, b[None, :], batch_f)


_ZROWS = 640
_LANES = 16


def _splat(ref, j):
    return plsc.load_gather(ref, [jnp.full((_LANES,), j, jnp.int32)])


def _edge_kernel(h_hbm, as_hbm, ad_hbm, src_hbm, dst_hbm, z_hbm, z1_hbm,
                 out_hbm, den_hbm,
                 as_l, ad_l, den_l, srcs0, srcs1, srcs2, dsts0, dsts1, dsts2,
                 ex_v, rows0, rows1, rows2, acc_sh, gs0, gs1, gs2,
                 ps0, ps1, ps2):
    cidx = lax.axis_index("c")
    sid = lax.axis_index("s")
    wid = sid * NC + cidx
    rows = (rows0, rows1, rows2)
    srcs = (srcs0, srcs1, srcs2)
    dsts = (dsts0, dsts1, dsts2)
    gsems = (gs0, gs1, gs2)
    psems = (ps0, ps1, ps2)

    pltpu.sync_copy(z_hbm, acc_sh.at[pl.ds(sid * _ZROWS, _ZROWS)])

    pltpu.sync_copy(as_hbm, as_l)
    pltpu.sync_copy(ad_hbm, ad_l)
    pltpu.sync_copy(z1_hbm, den_l)
    plsc.subcore_barrier()

    lanes = lax.broadcasted_iota(jnp.int32, (_LANES,), 0)

    def valid(c):
        return (wid + c * NW) < NCHUNK

    def base(c):
        return (wid + c * NW) * KE

    def pf_issue(c, slot):
        @pl.when(valid(c))
        def _():
            pltpu.async_copy(src_hbm.at[pl.ds(base(c), KE)], srcs[slot],
                             psems[slot])
            pltpu.async_copy(dst_hbm.at[pl.ds(base(c), KE)], dsts[slot],
                             psems[slot])

    def pf_wait(c, slot):
        @pl.when(valid(c))
        def _():
            pltpu.make_async_copy(src_hbm.at[pl.ds(base(c), KE)],
                                  srcs[slot], psems[slot]).wait()
            pltpu.make_async_copy(dst_hbm.at[pl.ds(base(c), KE)],
                                  dsts[slot], psems[slot]).wait()

    def g_issue(c, slot):
        @pl.when(valid(c))
        def _():
            pltpu.async_copy(h_hbm.at[srcs[slot]], rows[slot], gsems[slot])

    def g_wait(c, slot):
        @pl.when(valid(c))
        def _():
            pltpu.make_async_copy(h_hbm.at[srcs[slot]], rows[slot],
                                  gsems[slot]).wait()

    pf_issue(0, 0)
    pf_issue(1, 1)
    pf_issue(2, 2)
    pf_wait(0, 0)
    g_issue(0, 0)
    pf_wait(1, 1)
    g_issue(1, 1)

    def compute(c, slot):
        @pl.when(valid(c))
        def _():
            for g in range(KE // _LANES):
                s16 = srcs[slot][pl.ds(g * _LANES, _LANES)]
                d16 = dsts[slot][pl.ds(g * _LANES, _LANES)]
                e = (plsc.load_gather(as_l, [s16])
                     + plsc.load_gather(ad_l, [d16]))
                e = jnp.where(e > 0, e, NEG_SLOPE * e)
                ex16 = jnp.exp(e)
                ex_v[pl.ds(g * _LANES, _LANES)] = ex16
                s = jnp.zeros((_LANES,), jnp.float32)
                first = jnp.full((_LANES,), _LANES, jnp.int32)
                for j in range(_LANES):
                    dj = _splat(dsts[slot], g * _LANES + j)
                    vj = _splat(ex_v, g * _LANES + j)
                    eq = d16 == dj
                    s = s + jnp.where(eq, vj, 0.0)
                    first = jnp.minimum(
                        first, jnp.where(eq, j, _LANES).astype(jnp.int32))
                plsc.addupdate_scatter(den_l, [d16], s, mask=first == lanes)
            rv = rows[slot]
            for j in range(KE):
                m = _splat(ex_v, j)
                for k in range(C // _LANES):
                    rv[j, pl.ds(k * _LANES, _LANES)] = (
                        rv[j, pl.ds(k * _LANES, _LANES)] * m)
            pltpu.sync_copy(rows[slot], acc_sh.at[dsts[slot]], add=True)

    def step(k, carry):
        for b in range(3):
            c = 3 * k + b
            s2 = (b + 2) % 3
            pf_wait(c + 2, s2)
            g_issue(c + 2, s2)
            g_wait(c, b)
            compute(c, b)
            pf_issue(c + 3, b)
        return carry

    lax.fori_loop(0, (CPW + 2) // 3, step, 0)

    pltpu.sync_copy(den_l, den_hbm.at[pl.ds(wid * NP, NP)])
    plsc.subcore_barrier()

    pltpu.sync_copy(acc_sh.at[pl.ds(sid * _ZROWS, _ZROWS)],
                    out_hbm.at[cidx, pl.ds(sid * _ZROWS, _ZROWS)])


def _edge_layer(h, a_src, a_dst, src, dst, zeros_slab, zeros1):
    mesh = plsc.VectorSubcoreMesh(core_axis_name="c", subcore_axis_name="s")
    f = pl.kernel(
        _edge_kernel,
        out_type=[
            jax.ShapeDtypeStruct((NC, NP, C), jnp.float32),
            jax.ShapeDtypeStruct((NW * NP,), jnp.float32),
        ],
        mesh=mesh,
        scratch_types=[
            pltpu.VMEM((NP,), jnp.float32),
            pltpu.VMEM((NP,), jnp.float32),
            pltpu.VMEM((NP,), jnp.float32),
            pltpu.VMEM((KE,), jnp.int32),
            pltpu.VMEM((KE,), jnp.int32),
            pltpu.VMEM((KE,), jnp.int32),
            pltpu.VMEM((KE,), jnp.int32),
            pltpu.VMEM((KE,), jnp.int32),
            pltpu.VMEM((KE,), jnp.int32),
            pltpu.VMEM((KE,), jnp.float32),
            pltpu.VMEM((KE, C), jnp.float32),
            pltpu.VMEM((KE, C), jnp.float32),
            pltpu.VMEM((KE, C), jnp.float32),
            pltpu.VMEM_SHARED((NP, C), jnp.float32),
            pltpu.SemaphoreType.DMA,
            pltpu.SemaphoreType.DMA,
            pltpu.SemaphoreType.DMA,
            pltpu.SemaphoreType.DMA,
            pltpu.SemaphoreType.DMA,
            pltpu.SemaphoreType.DMA,
        ],
        compiler_params=pltpu.CompilerParams(needs_layout_passes=False),
    )
    acc, den_flat = f(h, a_src, a_dst, src, dst, zeros_slab, zeros1)
    return acc, den_flat.reshape(NW, NP)


def kernel(x, edge_index, batch,
           W_src1, W_dst1, a_src1, a_dst1, b1,
           W_src2, W_dst2, a_src2, a_dst2, b2,
           W_src3, W_dst3, a_src3, a_dst3, b3):
    src = edge_index[0]
    dst = edge_index[1]
    batch_f = jnp.concatenate(
        [batch.astype(jnp.float32),
         jnp.full((NP - N,), -1.0, jnp.float32)])[:, None]
    x = jnp.concatenate([x, jnp.zeros((NP - N, C), x.dtype)], axis=0)
    zeros_slab = jnp.zeros((_ZROWS, C), jnp.float32)
    zeros1 = jnp.zeros((NP,), jnp.float32)

    wd1 = W_dst1 @ a_dst1
    wd2 = W_dst2 @ a_dst2
    wd3 = W_dst3 @ a_dst3

    h, a_s, a_d = _dense_first(x, W_src1, a_src1, wd1)
    acc, den = _edge_layer(h, a_s[:, 0], a_d[:, 0], src, dst, zeros_slab, zeros1)
    h, a_s, a_d = _dense_mid(acc, den, b1, W_src2, a_src2, wd2)
    acc, den = _edge_layer(h, a_s[:, 0], a_d[:, 0], src, dst, zeros_slab, zeros1)
    h, a_s, a_d = _dense_mid(acc, den, b2, W_src3, a_src3, wd3)
    acc, den = _edge_layer(h, a_s[:, 0], a_d[:, 0], src, dst, zeros_slab, zeros1)
    return _pool(acc, den, b3, batch_f)


# speedup vs baseline: 1.0627x; 1.0627x over previous
import functools

import jax
import jax.numpy as jnp
from jax import lax
from jax.experimental import pallas as pl
from jax.experimental.pallas import tpu as pltpu
from jax.experimental.pallas import tpu_sc as plsc

N = 10000
NP = 10240
E = 320000
C = 128
G = 64
NEG_SLOPE = 0.2

BN = 2048
NC = 2
NS = 16
NW = NC * NS
KE = 64
NCHUNK = E // KE
CPW = -(-NCHUNK // NW)

_HIGH = jax.lax.Precision.HIGHEST


def _dense_first_body(x_ref, w_ref, att_ref, wd_ref, h_ref, as_ref, ad_ref):
    x = x_ref[...]
    h = jnp.dot(x, w_ref[...], preferred_element_type=jnp.float32,
                precision=_HIGH)
    h_ref[...] = h
    as_ref[...] = jnp.dot(h, att_ref[...], preferred_element_type=jnp.float32,
                          precision=_HIGH)
    ad_ref[...] = jnp.dot(x, wd_ref[...], preferred_element_type=jnp.float32,
                          precision=_HIGH)


def _epilogue(acc_ref, den_ref, b_ref, i):
    acc = acc_ref[0] + acc_ref[1]
    den = jnp.sum(den_ref[:, pl.ds(i * BN, BN)], axis=0)
    return jnp.maximum(acc / (den[:, None] + 1e-16) + b_ref[...], 0.0)


def _dense_mid_body(acc_ref, den_ref, b_ref, w_ref, att_ref, wd_ref,
                    h_ref, as_ref, ad_ref):
    x = _epilogue(acc_ref, den_ref, b_ref, pl.program_id(0))
    h = jnp.dot(x, w_ref[...], preferred_element_type=jnp.float32,
                precision=_HIGH)
    h_ref[...] = h
    as_ref[...] = jnp.dot(h, att_ref[...], preferred_element_type=jnp.float32,
                          precision=_HIGH)
    ad_ref[...] = jnp.dot(x, wd_ref[...], preferred_element_type=jnp.float32,
                          precision=_HIGH)


def _pool_body(acc_ref, den_ref, b_ref, batch_ref, out_ref, cnt_ref):
    i = pl.program_id(0)
    h = _epilogue(acc_ref, den_ref, b_ref, i)
    bids = batch_ref[...]
    iota = lax.broadcasted_iota(jnp.int32, (BN, G), 1).astype(jnp.float32)
    mask = (iota == bids).astype(jnp.float32)
    dnums = (((0,), (0,)), ((), ()))
    part = lax.dot_general(mask, h, dnums, preferred_element_type=jnp.float32,
                           precision=_HIGH)
    pcnt = lax.dot_general(mask, jnp.ones((BN, 1), jnp.float32), dnums,
                           preferred_element_type=jnp.float32,
                           precision=_HIGH)

    @pl.when(i == 0)
    def _():
        out_ref[...] = jnp.zeros_like(out_ref)
        cnt_ref[...] = jnp.zeros_like(cnt_ref)

    out_ref[...] += part
    cnt_ref[...] += pcnt

    @pl.when(i == pl.num_programs(0) - 1)
    def _():
        out_ref[...] = out_ref[...] / jnp.clip(cnt_ref[...], 1.0, None)


def _dense_first(x, W, att, wd):
    return pl.pallas_call(
        _dense_first_body,
        grid=(NP // BN,),
        in_specs=[
            pl.BlockSpec((BN, C), lambda i: (i, 0)),
            pl.BlockSpec((C, C), lambda i: (0, 0)),
            pl.BlockSpec((C, 1), lambda i: (0, 0)),
            pl.BlockSpec((C, 1), lambda i: (0, 0)),
        ],
        out_specs=[
            pl.BlockSpec((BN, C), lambda i: (i, 0)),
            pl.BlockSpec((BN, 1), lambda i: (i, 0)),
            pl.BlockSpec((BN, 1), lambda i: (i, 0)),
        ],
        out_shape=[
            jax.ShapeDtypeStruct((NP, C), jnp.float32),
            jax.ShapeDtypeStruct((NP, 1), jnp.float32),
            jax.ShapeDtypeStruct((NP, 1), jnp.float32),
        ],
    )(x, W, att[:, None], wd[:, None])


def _dense_mid(acc, den, b, W, att, wd):
    return pl.pallas_call(
        _dense_mid_body,
        grid=(NP // BN,),
        in_specs=[
            pl.BlockSpec((NC, BN, C), lambda i: (0, i, 0)),
            pl.BlockSpec((NW, NP), lambda i: (0, 0)),
            pl.BlockSpec((1, C), lambda i: (0, 0)),
            pl.BlockSpec((C, C), lambda i: (0, 0)),
            pl.BlockSpec((C, 1), lambda i: (0, 0)),
            pl.BlockSpec((C, 1), lambda i: (0, 0)),
        ],
        out_specs=[
            pl.BlockSpec((BN, C), lambda i: (i, 0)),
            pl.BlockSpec((BN, 1), lambda i: (i, 0)),
            pl.BlockSpec((BN, 1), lambda i: (i, 0)),
        ],
        out_shape=[
            jax.ShapeDtypeStruct((NP, C), jnp.float32),
            jax.ShapeDtypeStruct((NP, 1), jnp.float32),
            jax.ShapeDtypeStruct((NP, 1), jnp.float32),
        ],
    )(acc, den, b[None, :], W, att[:, None], wd[:, None])


def _pool(acc, den, b, batch_f):
    return pl.pallas_call(
        _pool_body,
        grid=(NP // BN,),
        in_specs=[
            pl.BlockSpec((NC, BN, C), lambda i: (0, i, 0)),
            pl.BlockSpec((NW, NP), lambda i: (0, 0)),
            pl.BlockSpec((1, C), lambda i: (0, 0)),
            pl.BlockSpec((BN, 1), lambda i: (i, 0)),
        ],
        out_specs=pl.BlockSpec((G, C), lambda i: (0, 0)),
        out_shape=jax.ShapeDtypeStruct((G, C), jnp.float32),
        scratch_shapes=[pltpu.VMEM((G, 1), jnp.float32)],
    )(acc, den, b[None, :], batch_f)


_ZROWS = 640
_LANES = 16


def _splat(ref, j):
    return plsc.load_gather(ref, [jnp.full((_LANES,), j, jnp.int32)])


def _edge_kernel(h_hbm, as_hbm, ad_hbm, src_hbm, dst_hbm, z_hbm, z1_hbm,
                 out_hbm, den_hbm,
                 as_l, ad_l, den_l, srcs0, srcs1, dsts0, dsts1, ex_v,
                 rows0, rows1, acc_sh, gsem, ps0, ps1):
    cidx = lax.axis_index("c")
    sid = lax.axis_index("s")
    wid = sid * NC + cidx
    rows = (rows0, rows1)
    srcs = (srcs0, srcs1)
    dsts = (dsts0, dsts1)
    psems = (ps0, ps1)

    pltpu.sync_copy(z_hbm, acc_sh.at[pl.ds(sid * _ZROWS, _ZROWS)])

    pltpu.sync_copy(as_hbm, as_l)
    pltpu.sync_copy(ad_hbm, ad_l)
    pltpu.sync_copy(z1_hbm, den_l)
    plsc.subcore_barrier()

    lanes = lax.broadcasted_iota(jnp.int32, (_LANES,), 0)

    def valid(c):
        return (wid + c * NW) < NCHUNK

    def base(c):
        return (wid + c * NW) * KE

    def pf_issue(c, slot):
        @pl.when(valid(c))
        def _():
            pltpu.async_copy(src_hbm.at[pl.ds(base(c), KE)], srcs[slot],
                             psems[slot])
            pltpu.async_copy(dst_hbm.at[pl.ds(base(c), KE)], dsts[slot],
                             psems[slot])

    def pf_wait(c, slot):
        @pl.when(valid(c))
        def _():
            pltpu.make_async_copy(src_hbm.at[pl.ds(base(c), KE)],
                                  srcs[slot], psems[slot]).wait()
            pltpu.make_async_copy(dst_hbm.at[pl.ds(base(c), KE)],
                                  dsts[slot], psems[slot]).wait()

    def g_issue(c, slot, ri):
        @pl.when(valid(c))
        def _():
            pltpu.async_copy(h_hbm.at[srcs[slot]], rows[ri], gsem)

    def g_wait(c, slot, ri):
        @pl.when(valid(c))
        def _():
            pltpu.make_async_copy(h_hbm.at[srcs[slot]], rows[ri],
                                  gsem).wait()


    pf_issue(0, 0)
    pf_issue(1, 1)
    pf_wait(0, 0)
    g_issue(0, 0, 0)

    def compute(c, slot, ri):
        @pl.when(valid(c))
        def _():
            for g in range(KE // _LANES):
                s16 = srcs[slot][pl.ds(g * _LANES, _LANES)]
                d16 = dsts[slot][pl.ds(g * _LANES, _LANES)]
                e = (plsc.load_gather(as_l, [s16])
                     + plsc.load_gather(ad_l, [d16]))
                e = jnp.where(e > 0, e, NEG_SLOPE * e)
                ex16 = jnp.exp(e)
                ex_v[pl.ds(g * _LANES, _LANES)] = ex16
                s = jnp.zeros((_LANES,), jnp.float32)
                first = jnp.full((_LANES,), _LANES, jnp.int32)
                for j in range(_LANES):
                    dj = _splat(dsts[slot], g * _LANES + j)
                    vj = _splat(ex_v, g * _LANES + j)
                    eq = d16 == dj
                    s = s + jnp.where(eq, vj, 0.0)
                    first = jnp.minimum(
                        first, jnp.where(eq, j, _LANES).astype(jnp.int32))
                plsc.addupdate_scatter(den_l, [d16], s, mask=first == lanes)
            rv = rows[ri]
            for j in range(KE):
                m = _splat(ex_v, j)
                for k in range(C // _LANES):
                    rv[j, pl.ds(k * _LANES, _LANES)] = (
                        rv[j, pl.ds(k * _LANES, _LANES)] * m)
            pltpu.sync_copy(rows[ri], acc_sh.at[dsts[slot]], add=True)

    def step(k, carry):
        for b in range(2):
            c = 2 * k + b
            ri = b
            pf_wait(c + 1, 1 - b)
            g_wait(c, b, ri)
            g_issue(c + 1, 1 - b, 1 - ri)
            compute(c, b, ri)
            pf_issue(c + 2, b)
        return carry

    lax.fori_loop(0, (CPW + 2) // 2, step, 0)

    pltpu.sync_copy(den_l, den_hbm.at[pl.ds(wid * NP, NP)])
    plsc.subcore_barrier()

    pltpu.sync_copy(acc_sh.at[pl.ds(sid * _ZROWS, _ZROWS)],
                    out_hbm.at[cidx, pl.ds(sid * _ZROWS, _ZROWS)])


def _edge_layer(h, a_src, a_dst, src, dst, zeros_slab, zeros1):
    mesh = plsc.VectorSubcoreMesh(core_axis_name="c", subcore_axis_name="s")
    f = pl.kernel(
        _edge_kernel,
        out_type=[
            jax.ShapeDtypeStruct((NC, NP, C), jnp.float32),
            jax.ShapeDtypeStruct((NW * NP,), jnp.float32),
        ],
        mesh=mesh,
        scratch_types=[
            pltpu.VMEM((NP,), jnp.float32),
            pltpu.VMEM((NP,), jnp.float32),
            pltpu.VMEM((NP,), jnp.float32),
            pltpu.VMEM((KE,), jnp.int32),
            pltpu.VMEM((KE,), jnp.int32),
            pltpu.VMEM((KE,), jnp.int32),
            pltpu.VMEM((KE,), jnp.int32),
            pltpu.VMEM((KE,), jnp.float32),
            pltpu.VMEM((KE, C), jnp.float32),
            pltpu.VMEM((KE, C), jnp.float32),
            pltpu.VMEM_SHARED((NP, C), jnp.float32),
            pltpu.SemaphoreType.DMA,
            pltpu.SemaphoreType.DMA,
            pltpu.SemaphoreType.DMA,
        ],
        compiler_params=pltpu.CompilerParams(needs_layout_passes=False),
    )
    acc, den_flat = f(h, a_src, a_dst, src, dst, zeros_slab, zeros1)
    return acc, den_flat.reshape(NW, NP)


def kernel(x, edge_index, batch,
           W_src1, W_dst1, a_src1, a_dst1, b1,
           W_src2, W_dst2, a_src2, a_dst2, b2,
           W_src3, W_dst3, a_src3, a_dst3, b3):
    src = edge_index[0]
    dst = edge_index[1]
    batch_f = jnp.concatenate(
        [batch.astype(jnp.float32),
         jnp.full((NP - N,), -1.0, jnp.float32)])[:, None]
    x = jnp.concatenate([x, jnp.zeros((NP - N, C), x.dtype)], axis=0)
    zeros_slab = jnp.zeros((_ZROWS, C), jnp.float32)
    zeros1 = jnp.zeros((NP,), jnp.float32)

    wd1 = W_dst1 @ a_dst1
    wd2 = W_dst2 @ a_dst2
    wd3 = W_dst3 @ a_dst3

    h, a_s, a_d = _dense_first(x, W_src1, a_src1, wd1)
    acc, den = _edge_layer(h, a_s[:, 0], a_d[:, 0], src, dst, zeros_slab, zeros1)
    h, a_s, a_d = _dense_mid(acc, den, b1, W_src2, a_src2, wd2)
    acc, den = _edge_layer(h, a_s[:, 0], a_d[:, 0], src, dst, zeros_slab, zeros1)
    h, a_s, a_d = _dense_mid(acc, den, b2, W_src3, a_src3, wd3)
    acc, den = _edge_layer(h, a_s[:, 0], a_d[:, 0], src, dst, zeros_slab, zeros1)
    return _pool(acc, den, b3, batch_f)
